# baseline (speedup 1.0000x reference)
import functools

import jax
import jax.numpy as jnp
from jax import lax
from jax.experimental import pallas as pl
from jax.experimental.pallas import tpu as pltpu
from jax.experimental.pallas import tpu_sc as plsc


_ROWS = 16


def _rank_body(s_ref, s_smem, rank_ref):
  bsz, rows, cols = s_ref.shape
  ii = (lax.broadcasted_iota(jnp.int32, (rows, cols), 0) * cols +
        lax.broadcasted_iota(jnp.int32, (rows, cols), 1))
  svs = [s_ref[b] for b in range(bsz)]

  unroll = 8

  def jstep(j0, ranks):
    out = list(ranks)
    for u in range(unroll):
      j = j0 * unroll + u
      gtj = ii > j
      for b in range(bsz):
        sj = s_smem[b, j]
        beat = (sj > svs[b]) | ((sj == svs[b]) & gtj)
        out[b] = out[b] + beat.astype(jnp.int32)
    return tuple(out)

  ranks = lax.fori_loop(
      0, rows * cols // unroll, jstep,
      tuple(jnp.zeros((rows, cols), jnp.int32) for _ in range(bsz)))
  for b in range(bsz):
    rank_ref[b] = ranks[b]


def _rank_scores(scores):
  b, n = scores.shape
  rank = pl.pallas_call(
      _rank_body,
      in_specs=[
          pl.BlockSpec(memory_space=pltpu.VMEM),
          pl.BlockSpec(memory_space=pltpu.SMEM),
      ],
      out_shape=jax.ShapeDtypeStruct((b, _ROWS, n // _ROWS), jnp.int32),
  )(scores.reshape(b, _ROWS, n // _ROWS), scores)
  return rank.reshape(b, n)


_LANES = 16


def _col_gather(src, cidx, dst, ch, k):
  qunroll = 8

  def qstep(q, carry):
    for u in range(qunroll):
      q0 = pl.multiple_of((q * qunroll + u) * _LANES, _LANES)
      cvec = cidx[pl.ds(q0, _LANES)]
      got = [
          plsc.load_gather(src, [jnp.full((_LANES,), j, jnp.int32), cvec])
          for j in range(ch)
      ]
      for j in range(ch):
        dst[j, pl.ds(q0, _LANES)] = got[j]
    return carry

  lax.fori_loop(0, k // (_LANES * qunroll), qstep, 0)


def _scale_rows(buf, vals, row0, ch, d):
  vvecs = [
      plsc.load_gather(vals, [jnp.full((_LANES,), row0 + j, jnp.int32)])
      for j in range(ch)
  ]

  def qstep(q, carry):
    q0 = pl.multiple_of(q * _LANES, _LANES)
    for j in range(ch):
      buf[j, pl.ds(q0, _LANES)] = buf[j, pl.ds(q0, _LANES)] * vvecs[j]
    return carry

  lax.fori_loop(0, d // _LANES, qstep, 0)


def _make_sc_gather(b, n, d, k):
  info = plsc.get_sparse_core_info()
  ncores, nsub = info.num_cores, info.num_subcores
  nworkers = ncores * nsub
  assert k % (nworkers // b) == 0 and nworkers % b == 0
  wpb = nworkers // b
  rpw = k // wpb
  ch = 16
  nchunks = rpw // ch
  mesh = plsc.VectorSubcoreMesh(
      core_axis_name="c", subcore_axis_name="s",
      num_cores=ncores, num_subcores=nsub)

  @functools.partial(
      pl.kernel,
      mesh=mesh,
      compiler_params=pltpu.CompilerParams(
          use_tc_tiling_on_sc=True, needs_layout_passes=False),
      out_type=[
          jax.ShapeDtypeStruct((b, k, k), jnp.float32),
          jax.ShapeDtypeStruct((b, k, k), jnp.float32),
          jax.ShapeDtypeStruct((b, k, n), jnp.float32),
          jax.ShapeDtypeStruct((b, k, d), jnp.float32),
      ],
      scratch_types=[
          pltpu.VMEM((n,), jnp.int32),
          pltpu.VMEM((n,), jnp.float32),
          pltpu.VMEM((k,), jnp.int32),
          pltpu.VMEM((k,), jnp.float32),
          pltpu.VMEM((ch, n), jnp.float32),
          pltpu.VMEM((ch, n), jnp.float32),
          pltpu.VMEM((ch, k), jnp.float32),
          pltpu.VMEM((ch, k), jnp.float32),
          pltpu.SemaphoreType.DMA,
          pltpu.SemaphoreType.DMA,
          pltpu.SemaphoreType.DMA,
          pltpu.SemaphoreType.DMA,
          pltpu.SemaphoreType.DMA,
          pltpu.SemaphoreType.DMA,
      ],
  )
  def sc_gather(g1, g2, g3, h, rank, scores, gsec, gsent, gmask, newh,
                rank_v, s_v, cidx_v, vals_v, wide0, wide1, nar0, nar1,
                sem_in0, sem_in1, sem_out0, sem_out1, sem_wout0, sem_wout1):
    wid = lax.axis_index("s") * ncores + lax.axis_index("c")
    bi = wid // wpb
    r0 = (wid % wpb) * rpw
    pltpu.sync_copy(rank.at[bi], rank_v)
    pltpu.sync_copy(scores.at[bi], s_v)

    def qstep(q, carry):
      q0 = pl.multiple_of(q * _LANES, _LANES)
      rk = rank_v[pl.ds(q0, _LANES)]
      sv = s_v[pl.ds(q0, _LANES)]
      iv = lax.broadcasted_iota(jnp.int32, (_LANES,), 0) + q0
      m = rk < k
      rkc = jnp.where(m, rk, 0)
      plsc.store_scatter(cidx_v, [rkc], iv, mask=m)
      plsc.store_scatter(vals_v, [rkc], sv, mask=m)
      return carry

    lax.fori_loop(0, n // _LANES, qstep, 0)

    wides = (wide0, wide1)
    nars = (nar0, nar1)
    ins = (sem_in0, sem_in1)
    outs = (sem_out0, sem_out1)
    wouts = (sem_wout0, sem_wout1)

    def rvec(c):
      return cidx_v[pl.ds(r0 + c * ch, ch)]

    def meta(t):
      c, r = divmod(t, 3)
      return ((g1, gsec, c, False), (g3, gmask, c, True),
              (g2, gsent, c, False))[r]

    total = 3 * nchunks
    in_h = [None, None]
    out_h = [None, None]
    wout_h = [None, None]

    def issue(t):
      s = t % 2
      if wout_h[s] is not None:
        wout_h[s].wait()
        wout_h[s] = None
      gsrc, _, c, _ = meta(t)
      in_h[s] = pltpu.async_copy(gsrc.at[bi].at[rvec(c)], wides[s], ins[s])

    for t in range(total):
      s = t % 2
      if t == 0:
        issue(0)
      if t + 1 < total:
        issue(t + 1)
      in_h[s].wait()
      _, gdst, c, is_copy = meta(t)
      orows = pl.ds(r0 + c * ch, ch)
      if is_copy:
        wout_h[s] = pltpu.async_copy(wides[s], gdst.at[bi, orows], wouts[s])
      else:
        if out_h[s] is not None:
          out_h[s].wait()
        _col_gather(wides[s], cidx_v, nars[s], ch, k)
        out_h[s] = pltpu.async_copy(nars[s], gdst.at[bi, orows], outs[s])
    for s in (0, 1):
      if wout_h[s] is not None:
        wout_h[s].wait()

    def issue_h(t):
      s = t % 2
      if out_h[s] is not None:
        out_h[s].wait()
        out_h[s] = None
      in_h[s] = pltpu.async_copy(h.at[bi].at[rvec(t)], nars[s], ins[s])

    for t in range(nchunks):
      s = t % 2
      if t == 0:
        issue_h(0)
      if t + 1 < nchunks:
        issue_h(t + 1)
      in_h[s].wait()
      _scale_rows(nars[s], vals_v, r0 + t * ch, ch, d)
      out_h[s] = pltpu.async_copy(
          nars[s], newh.at[bi, pl.ds(r0 + t * ch, ch)], outs[s])

    out_h[0].wait()
    out_h[1].wait()

  return sc_gather


def kernel(g1, g2, g3, h, section_feature):
  b, n, d = h.shape
  k = max(2, n // 2)
  weights = jnp.matmul(h, jnp.swapaxes(section_feature, 1, 2))[..., 0]
  scores = jax.nn.sigmoid(weights)
  rank = _rank_scores(scores)
  gsec, gsent, gmask, newh = _make_sc_gather(b, n, d, k)(
      g1, g2, g3, h, rank, scores)
  return (gsec, gsent, gmask, newh)


# speedup vs baseline: 1.0241x; 1.0241x over previous
import functools

import jax
import jax.numpy as jnp
from jax import lax
from jax.experimental import pallas as pl
from jax.experimental.pallas import tpu as pltpu
from jax.experimental.pallas import tpu_sc as plsc


_ROWS = 16


def _rank_body(s_ref, s_smem, rank_ref):
  bsz, rows, cols = s_ref.shape
  ii = (lax.broadcasted_iota(jnp.int32, (rows, cols), 0) * cols +
        lax.broadcasted_iota(jnp.int32, (rows, cols), 1))
  svs = [s_ref[b] for b in range(bsz)]

  unroll = 8

  def jstep(j0, ranks):
    out = list(ranks)
    for u in range(unroll):
      j = j0 * unroll + u
      gtj = ii > j
      for b in range(bsz):
        sj = s_smem[b, j]
        beat = (sj > svs[b]) | ((sj == svs[b]) & gtj)
        out[b] = out[b] + beat.astype(jnp.int32)
    return tuple(out)

  ranks = lax.fori_loop(
      0, rows * cols // unroll, jstep,
      tuple(jnp.zeros((rows, cols), jnp.int32) for _ in range(bsz)))
  for b in range(bsz):
    rank_ref[b] = ranks[b]


def _rank_scores(scores):
  b, n = scores.shape
  rank = pl.pallas_call(
      _rank_body,
      in_specs=[
          pl.BlockSpec(memory_space=pltpu.VMEM),
          pl.BlockSpec(memory_space=pltpu.SMEM),
      ],
      out_shape=jax.ShapeDtypeStruct((b, _ROWS, n // _ROWS), jnp.int32),
  )(scores.reshape(b, _ROWS, n // _ROWS), scores)
  return rank.reshape(b, n)


_LANES = 16


def _col_gather(src, cidx, dst, ch, k):
  qunroll = 4

  def qstep(q, carry):
    for u in range(qunroll):
      q0 = pl.multiple_of((q * qunroll + u) * _LANES, _LANES)
      cvec = cidx[pl.ds(q0, _LANES)]
      got = [
          plsc.load_gather(src, [jnp.full((_LANES,), j, jnp.int32), cvec])
          for j in range(ch)
      ]
      for j in range(ch):
        dst[j, pl.ds(q0, _LANES)] = got[j]
    return carry

  lax.fori_loop(0, k // (_LANES * qunroll), qstep, 0)


def _scale_rows(buf, vals, row0, ch, d):
  vvecs = [
      plsc.load_gather(vals, [jnp.full((_LANES,), row0 + j, jnp.int32)])
      for j in range(ch)
  ]

  qunroll = 4

  def qstep(q, carry):
    for u in range(qunroll):
      q0 = pl.multiple_of((q * qunroll + u) * _LANES, _LANES)
      got = [buf[j, pl.ds(q0, _LANES)] * vvecs[j] for j in range(ch)]
      for j in range(ch):
        buf[j, pl.ds(q0, _LANES)] = got[j]
    return carry

  lax.fori_loop(0, d // (_LANES * qunroll), qstep, 0)


def _make_sc_gather(b, n, d, k):
  info = plsc.get_sparse_core_info()
  ncores, nsub = info.num_cores, info.num_subcores
  nworkers = ncores * nsub
  assert k % (nworkers // b) == 0 and nworkers % b == 0
  wpb = nworkers // b
  rpw = k // wpb
  ch = 16
  nchunks = rpw // ch
  mesh = plsc.VectorSubcoreMesh(
      core_axis_name="c", subcore_axis_name="s",
      num_cores=ncores, num_subcores=nsub)

  @functools.partial(
      pl.kernel,
      mesh=mesh,
      compiler_params=pltpu.CompilerParams(
          use_tc_tiling_on_sc=True, needs_layout_passes=False),
      out_type=[
          jax.ShapeDtypeStruct((b, k, k), jnp.float32),
          jax.ShapeDtypeStruct((b, k, k), jnp.float32),
          jax.ShapeDtypeStruct((b, k, n), jnp.float32),
          jax.ShapeDtypeStruct((b, k, d), jnp.float32),
      ],
      scratch_types=[
          pltpu.VMEM((n,), jnp.int32),
          pltpu.VMEM((n,), jnp.float32),
          pltpu.VMEM((k,), jnp.int32),
          pltpu.VMEM((k,), jnp.float32),
          pltpu.VMEM((ch, n), jnp.float32),
          pltpu.VMEM((ch, n), jnp.float32),
          pltpu.VMEM((ch, k), jnp.float32),
          pltpu.VMEM((ch, k), jnp.float32),
          pltpu.SemaphoreType.DMA,
          pltpu.SemaphoreType.DMA,
          pltpu.SemaphoreType.DMA,
          pltpu.SemaphoreType.DMA,
          pltpu.SemaphoreType.DMA,
          pltpu.SemaphoreType.DMA,
      ],
  )
  def sc_gather(g1, g2, g3, h, rank, scores, gsec, gsent, gmask, newh,
                rank_v, s_v, cidx_v, vals_v, wide0, wide1, nar0, nar1,
                sem_in0, sem_in1, sem_out0, sem_out1, sem_wout0, sem_wout1):
    wid = lax.axis_index("s") * ncores + lax.axis_index("c")
    bi = wid // wpb
    r0 = (wid % wpb) * rpw
    pltpu.sync_copy(rank.at[bi], rank_v)
    pltpu.sync_copy(scores.at[bi], s_v)

    def qstep(q, carry):
      q0 = pl.multiple_of(q * _LANES, _LANES)
      rk = rank_v[pl.ds(q0, _LANES)]
      sv = s_v[pl.ds(q0, _LANES)]
      iv = lax.broadcasted_iota(jnp.int32, (_LANES,), 0) + q0
      m = rk < k
      rkc = jnp.where(m, rk, 0)
      plsc.store_scatter(cidx_v, [rkc], iv, mask=m)
      plsc.store_scatter(vals_v, [rkc], sv, mask=m)
      return carry

    lax.fori_loop(0, n // _LANES, qstep, 0)

    wides = (wide0, wide1)
    nars = (nar0, nar1)
    ins = (sem_in0, sem_in1)
    outs = (sem_out0, sem_out1)
    wouts = (sem_wout0, sem_wout1)

    def rvec(c):
      return cidx_v[pl.ds(r0 + c * ch, ch)]

    def meta(t):
      c, r = divmod(t, 3)
      return ((g1, gsec, c, False), (g3, gmask, c, True),
              (g2, gsent, c, False))[r]

    total = 3 * nchunks
    in_h = [None, None]
    out_h = [None, None]
    wout_h = [None, None]

    def issue(t):
      s = t % 2
      if wout_h[s] is not None:
        wout_h[s].wait()
        wout_h[s] = None
      gsrc, _, c, _ = meta(t)
      in_h[s] = pltpu.async_copy(gsrc.at[bi].at[rvec(c)], wides[s], ins[s])

    for t in range(total):
      s = t % 2
      if t == 0:
        issue(0)
      if t + 1 < total:
        issue(t + 1)
      in_h[s].wait()
      _, gdst, c, is_copy = meta(t)
      orows = pl.ds(r0 + c * ch, ch)
      if is_copy:
        wout_h[s] = pltpu.async_copy(wides[s], gdst.at[bi, orows], wouts[s])
      else:
        if out_h[s] is not None:
          out_h[s].wait()
        _col_gather(wides[s], cidx_v, nars[s], ch, k)
        out_h[s] = pltpu.async_copy(nars[s], gdst.at[bi, orows], outs[s])
    for s in (0, 1):
      if wout_h[s] is not None:
        wout_h[s].wait()

    def issue_h(t):
      s = t % 2
      if out_h[s] is not None:
        out_h[s].wait()
        out_h[s] = None
      in_h[s] = pltpu.async_copy(h.at[bi].at[rvec(t)], nars[s], ins[s])

    for t in range(nchunks):
      s = t % 2
      if t == 0:
        issue_h(0)
      if t + 1 < nchunks:
        issue_h(t + 1)
      in_h[s].wait()
      _scale_rows(nars[s], vals_v, r0 + t * ch, ch, d)
      out_h[s] = pltpu.async_copy(
          nars[s], newh.at[bi, pl.ds(r0 + t * ch, ch)], outs[s])

    out_h[0].wait()
    out_h[1].wait()

  return sc_gather


def kernel(g1, g2, g3, h, section_feature):
  b, n, d = h.shape
  k = max(2, n // 2)
  weights = jnp.matmul(h, jnp.swapaxes(section_feature, 1, 2))[..., 0]
  scores = jax.nn.sigmoid(weights)
  rank = _rank_scores(scores)
  gsec, gsent, gmask, newh = _make_sc_gather(b, n, d, k)(
      g1, g2, g3, h, rank, scores)
  return (gsec, gsent, gmask, newh)


# speedup vs baseline: 1.0253x; 1.0012x over previous
import functools

import jax
import jax.numpy as jnp
from jax import lax
from jax.experimental import pallas as pl
from jax.experimental.pallas import tpu as pltpu
from jax.experimental.pallas import tpu_sc as plsc


_ROWS = 16


def _rank_body(s_ref, s_smem, rank_ref):
  bsz, rows, cols = s_ref.shape
  ii = (lax.broadcasted_iota(jnp.int32, (rows, cols), 0) * cols +
        lax.broadcasted_iota(jnp.int32, (rows, cols), 1))
  svs = [s_ref[b] for b in range(bsz)]

  unroll = 8

  def jstep(j0, ranks):
    out = list(ranks)
    for u in range(unroll):
      j = j0 * unroll + u
      gtj = ii > j
      for b in range(bsz):
        sj = s_smem[b, j]
        beat = (sj > svs[b]) | ((sj == svs[b]) & gtj)
        out[b] = out[b] + beat.astype(jnp.int32)
    return tuple(out)

  ranks = lax.fori_loop(
      0, rows * cols // unroll, jstep,
      tuple(jnp.zeros((rows, cols), jnp.int32) for _ in range(bsz)))
  for b in range(bsz):
    rank_ref[b] = ranks[b]


def _rank_scores(scores):
  b, n = scores.shape
  rank = pl.pallas_call(
      _rank_body,
      in_specs=[
          pl.BlockSpec(memory_space=pltpu.VMEM),
          pl.BlockSpec(memory_space=pltpu.SMEM),
      ],
      out_shape=jax.ShapeDtypeStruct((b, _ROWS, n // _ROWS), jnp.int32),
  )(scores.reshape(b, _ROWS, n // _ROWS), scores)
  return rank.reshape(b, n)


_LANES = 16


def _col_gather(src, cidx, dst, ch, k):
  qunroll = 4

  def qstep(q, carry):
    for u in range(qunroll):
      q0 = pl.multiple_of((q * qunroll + u) * _LANES, _LANES)
      cvec = cidx[pl.ds(q0, _LANES)]
      got = [
          plsc.load_gather(src, [jnp.full((_LANES,), j, jnp.int32), cvec])
          for j in range(ch)
      ]
      for j in range(ch):
        dst[j, pl.ds(q0, _LANES)] = got[j]
    return carry

  lax.fori_loop(0, k // (_LANES * qunroll), qstep, 0)


def _scale_rows(buf, vals, row0, ch, d):
  vvecs = [
      plsc.load_gather(vals, [jnp.full((_LANES,), row0 + j, jnp.int32)])
      for j in range(ch)
  ]

  qunroll = 4

  def qstep(q, carry):
    for u in range(qunroll):
      q0 = pl.multiple_of((q * qunroll + u) * _LANES, _LANES)
      got = [buf[j, pl.ds(q0, _LANES)] * vvecs[j] for j in range(ch)]
      for j in range(ch):
        buf[j, pl.ds(q0, _LANES)] = got[j]
    return carry

  lax.fori_loop(0, d // (_LANES * qunroll), qstep, 0)


def _make_sc_gather(b, n, d, k):
  info = plsc.get_sparse_core_info()
  ncores, nsub = info.num_cores, info.num_subcores
  nworkers = ncores * nsub
  assert k % (nworkers // b) == 0 and nworkers % b == 0
  wpb = nworkers // b
  rpw = k // wpb
  ch = 16
  nchunks = rpw // ch
  mesh = plsc.VectorSubcoreMesh(
      core_axis_name="c", subcore_axis_name="s",
      num_cores=ncores, num_subcores=nsub)

  @functools.partial(
      pl.kernel,
      mesh=mesh,
      compiler_params=pltpu.CompilerParams(
          use_tc_tiling_on_sc=True, needs_layout_passes=False),
      out_type=[
          jax.ShapeDtypeStruct((b, k, k), jnp.float32),
          jax.ShapeDtypeStruct((b, k, k), jnp.float32),
          jax.ShapeDtypeStruct((b, k, n), jnp.float32),
          jax.ShapeDtypeStruct((b, k, d), jnp.float32),
      ],
      scratch_types=[
          pltpu.VMEM((n,), jnp.int32),
          pltpu.VMEM((n,), jnp.float32),
          pltpu.VMEM((k,), jnp.int32),
          pltpu.VMEM((k,), jnp.float32),
          pltpu.VMEM((ch, n), jnp.float32),
          pltpu.VMEM((ch, n), jnp.float32),
          pltpu.VMEM((ch, k), jnp.float32),
          pltpu.VMEM((ch, k), jnp.float32),
          pltpu.SemaphoreType.DMA,
          pltpu.SemaphoreType.DMA,
          pltpu.SemaphoreType.DMA,
          pltpu.SemaphoreType.DMA,
          pltpu.SemaphoreType.DMA,
          pltpu.SemaphoreType.DMA,
      ],
  )
  def sc_gather(g1, g2, g3, h, rank, scores, gsec, gsent, gmask, newh,
                rank_v, s_v, cidx_v, vals_v, wide0, wide1, nar0, nar1,
                sem_in0, sem_in1, sem_out0, sem_out1, sem_wout0, sem_wout1):
    wid = lax.axis_index("s") * ncores + lax.axis_index("c")
    bi = wid // wpb
    r0 = (wid % wpb) * rpw
    pltpu.sync_copy(rank.at[bi], rank_v)
    pltpu.sync_copy(scores.at[bi], s_v)

    def qstep(q, carry):
      for u in range(4):
        q0 = pl.multiple_of((q * 4 + u) * _LANES, _LANES)
        rk = rank_v[pl.ds(q0, _LANES)]
        sv = s_v[pl.ds(q0, _LANES)]
        iv = lax.broadcasted_iota(jnp.int32, (_LANES,), 0) + q0
        m = rk < k
        rkc = jnp.where(m, rk, 0)
        plsc.store_scatter(cidx_v, [rkc], iv, mask=m)
        plsc.store_scatter(vals_v, [rkc], sv, mask=m)
      return carry

    lax.fori_loop(0, n // (_LANES * 4), qstep, 0)

    wides = (wide0, wide1)
    nars = (nar0, nar1)
    ins = (sem_in0, sem_in1)
    outs = (sem_out0, sem_out1)
    wouts = (sem_wout0, sem_wout1)

    def rvec(c):
      return cidx_v[pl.ds(r0 + c * ch, ch)]

    def meta(t):
      c, r = divmod(t, 3)
      return ((g1, gsec, c, False), (g3, gmask, c, True),
              (g2, gsent, c, False))[r]

    total = 3 * nchunks
    in_h = [None, None]
    out_h = [None, None]
    wout_h = [None, None]

    def issue(t):
      s = t % 2
      if wout_h[s] is not None:
        wout_h[s].wait()
        wout_h[s] = None
      gsrc, _, c, _ = meta(t)
      in_h[s] = pltpu.async_copy(gsrc.at[bi].at[rvec(c)], wides[s], ins[s])

    for t in range(total):
      s = t % 2
      if t == 0:
        issue(0)
      if t + 1 < total:
        issue(t + 1)
      in_h[s].wait()
      _, gdst, c, is_copy = meta(t)
      orows = pl.ds(r0 + c * ch, ch)
      if is_copy:
        wout_h[s] = pltpu.async_copy(wides[s], gdst.at[bi, orows], wouts[s])
      else:
        if out_h[s] is not None:
          out_h[s].wait()
        _col_gather(wides[s], cidx_v, nars[s], ch, k)
        out_h[s] = pltpu.async_copy(nars[s], gdst.at[bi, orows], outs[s])
    for s in (0, 1):
      if wout_h[s] is not None:
        wout_h[s].wait()

    def issue_h(t):
      s = t % 2
      if out_h[s] is not None:
        out_h[s].wait()
        out_h[s] = None
      in_h[s] = pltpu.async_copy(h.at[bi].at[rvec(t)], nars[s], ins[s])

    for t in range(nchunks):
      s = t % 2
      if t == 0:
        issue_h(0)
      if t + 1 < nchunks:
        issue_h(t + 1)
      in_h[s].wait()
      _scale_rows(nars[s], vals_v, r0 + t * ch, ch, d)
      out_h[s] = pltpu.async_copy(
          nars[s], newh.at[bi, pl.ds(r0 + t * ch, ch)], outs[s])

    out_h[0].wait()
    out_h[1].wait()

  return sc_gather


def kernel(g1, g2, g3, h, section_feature):
  b, n, d = h.shape
  k = max(2, n // 2)
  weights = jnp.matmul(h, jnp.swapaxes(section_feature, 1, 2))[..., 0]
  scores = jax.nn.sigmoid(weights)
  rank = _rank_scores(scores)
  gsec, gsent, gmask, newh = _make_sc_gather(b, n, d, k)(
      g1, g2, g3, h, rank, scores)
  return (gsec, gsent, gmask, newh)
